# baseline (speedup 1.0000x reference)
import jax
import jax.numpy as jnp
from jax import lax
from jax.experimental import pallas as pl
from jax.experimental.pallas import tpu as pltpu
from jax.experimental.pallas import tpu_sc as plsc

VOCAB_SIZE = 1_000_000
EMBED_DIM = 32
BATCH = 16384
HIST = 50
HIST_PAD = 56
LANE_PAD = 128
Q_TOTAL = BATCH * HIST_PAD
NUM_WORKERS = 32
Q_PER_W = Q_TOTAL // NUM_WORKERS
CQ = 1792
NCHUNK = Q_PER_W // CQ

TW = 1024
NBLK = VOCAB_SIZE // TW
TAIL = 512
V_COVERED = NBLK * TW + TAIL
TOUT_ROWS = TW * EMBED_DIM // LANE_PAD


def _transpose_block(tabt_hbm, out_hbm, buf_a, buf_b, v0, w):
    v0 = pl.multiple_of(v0, LANE_PAD)

    @pl.loop(0, EMBED_DIM // 8)
    def _rd(c8):
        c0 = pl.multiple_of(c8 * 8, 8)
        pltpu.sync_copy(tabt_hbm.at[pl.ds(c0, 8), pl.ds(v0, w)],
                        buf_a.at[pl.ds(c0, 8), pl.ds(0, w)])

    lanes = lax.iota(jnp.int32, 16)

    @pl.loop(0, w, unroll=8)
    def _tr(j):
        idx_j = jnp.broadcast_to(j, (16,)).astype(jnp.int32)
        for g in range(2):
            idx_c = lanes + g * 16
            vals = plsc.load_gather(buf_a, [idx_c, idx_j])
            flat = j * EMBED_DIM + g * 16
            buf_b[flat // LANE_PAD, pl.ds(flat % LANE_PAD, 16)] = vals

    out_rows = w * EMBED_DIM // LANE_PAD
    r0 = pl.multiple_of(v0 * EMBED_DIM // LANE_PAD, 8)
    pltpu.sync_copy(buf_b.at[pl.ds(0, out_rows), :],
                    out_hbm.at[pl.ds(r0, out_rows), :])


def _transpose_body(tabt_hbm, out_hbm, buf_a, buf_b, sem):
    wid = lax.axis_index("s") * 2 + lax.axis_index("c")

    @pl.loop(wid, NBLK, step=NUM_WORKERS)
    def _blk(blk):
        _transpose_block(tabt_hbm, out_hbm, buf_a, buf_b, blk * TW, TW)

    @pl.when(wid == NUM_WORKERS - 1)
    def _tail():
        _transpose_block(tabt_hbm, out_hbm, buf_a, buf_b, NBLK * TW, TAIL)


def _gather_body(idx_hbm, table_hbm, out_hbm, idx0, idx1, rows0, rows1,
                 sem0, sem1):
    wid = lax.axis_index("s") * 2 + lax.axis_index("c")
    base = wid * Q_PER_W

    def _wait(rows_v, sem):
        pltpu.make_async_copy(table_hbm.at[pl.ds(0, CQ)], rows_v, sem).wait()

    def _start(c, idx_v, rows_v, sem):
        pltpu.sync_copy(idx_hbm.at[pl.ds(base + c * CQ, CQ)], idx_v)
        pltpu.async_copy(table_hbm.at[idx_v], rows_v, sem)

    def _store(c, rows_v):
        pltpu.sync_copy(
            rows_v, out_hbm.at[pl.ds(base + c * CQ, CQ), pl.ds(0, EMBED_DIM)]
        )

    _start(0, idx0, rows0, sem0)

    @pl.loop(0, NCHUNK, step=2)
    def _chunk(c):
        _start(c + 1, idx1, rows1, sem1)
        _wait(rows0, sem0)
        _store(c, rows0)

        @pl.when(c + 2 < NCHUNK)
        def _():
            _start(c + 2, idx0, rows0, sem0)

        _wait(rows1, sem1)
        _store(c + 1, rows1)


@jax.jit
def _embed(token_ids, embeddings):
    mesh = plsc.VectorSubcoreMesh(core_axis_name="c", subcore_axis_name="s")

    transpose_kernel = pl.kernel(
        _transpose_body,
        out_type=jax.ShapeDtypeStruct(
            (VOCAB_SIZE * EMBED_DIM // LANE_PAD, LANE_PAD), jnp.float32
        ),
        mesh=mesh,
        scratch_types=[
            pltpu.VMEM((EMBED_DIM, TW), jnp.float32),
            pltpu.VMEM((TOUT_ROWS, LANE_PAD), jnp.float32),
            pltpu.SemaphoreType.DMA,
        ],
        compiler_params=pltpu.CompilerParams(needs_layout_passes=False),
    )
    table_rm = transpose_kernel(embeddings.T).reshape(VOCAB_SIZE, EMBED_DIM)
    tail = lax.slice(embeddings, (V_COVERED, 0), (VOCAB_SIZE, EMBED_DIM))
    table_rm = lax.dynamic_update_slice(table_rm, tail, (V_COVERED, 0))

    idx56 = jnp.concatenate(
        [token_ids, token_ids[:, : HIST_PAD - HIST]], axis=1
    ).astype(jnp.int32)
    idx_flat = idx56.reshape(-1)

    gather_kernel = pl.kernel(
        _gather_body,
        out_type=jax.ShapeDtypeStruct((Q_TOTAL, LANE_PAD), jnp.float32),
        mesh=mesh,
        scratch_types=[
            pltpu.VMEM((CQ,), jnp.int32),
            pltpu.VMEM((CQ,), jnp.int32),
            pltpu.VMEM((CQ, EMBED_DIM), jnp.float32),
            pltpu.VMEM((CQ, EMBED_DIM), jnp.float32),
            pltpu.SemaphoreType.DMA,
            pltpu.SemaphoreType.DMA,
        ],
        compiler_params=pltpu.CompilerParams(use_tc_tiling_on_sc=False),
    )
    padded = gather_kernel(idx_flat, table_rm)
    padded3 = padded.reshape(BATCH, HIST_PAD, LANE_PAD)
    return lax.slice(padded3, (0, 0, 0), (BATCH, HIST, EMBED_DIM))


def kernel(token_ids, embeddings):
    return _embed(token_ids, embeddings)


# speedup vs baseline: 1.5560x; 1.5560x over previous
import jax
import jax.numpy as jnp
from jax import lax
from jax.experimental import pallas as pl
from jax.experimental.pallas import tpu as pltpu
from jax.experimental.pallas import tpu_sc as plsc

VOCAB_SIZE = 1_000_000
EMBED_DIM = 32
BATCH = 16384
HIST = 50
HIST_PAD = 56
LANE_PAD = 128
Q_TOTAL = BATCH * HIST_PAD
NUM_WORKERS = 32
Q_PER_W = Q_TOTAL // NUM_WORKERS
CQ = 1792
NCHUNK = Q_PER_W // CQ

TW = 1024
NBLK = VOCAB_SIZE // TW
TAIL = 512
V_COVERED = NBLK * TW + TAIL
TOUT_ROWS = TW * EMBED_DIM // LANE_PAD
ROWS_TOTAL = VOCAB_SIZE * EMBED_DIM // LANE_PAD


def _transpose_block(tabt_hbm, out_hbm, buf_a, buf_b, v0, w):
    v0 = pl.multiple_of(v0, LANE_PAD)

    @pl.loop(0, EMBED_DIM // 8)
    def _rd(c8):
        c0 = pl.multiple_of(c8 * 8, 8)
        pltpu.sync_copy(tabt_hbm.at[pl.ds(c0, 8), pl.ds(v0, w)],
                        buf_a.at[pl.ds(c0, 8), pl.ds(0, w)])

    lanes = lax.iota(jnp.int32, 16)

    @pl.loop(0, w // 4, unroll=8)
    def _tr(j4):
        row = j4
        for dj in range(4):
            idx_j = jnp.broadcast_to(j4 * 4 + dj, (16,)).astype(jnp.int32)
            for g in range(2):
                idx_c = lanes + g * 16
                vals = plsc.load_gather(buf_a, [idx_c, idx_j])
                buf_b[row, pl.ds(dj * EMBED_DIM + g * 16, 16)] = vals

    out_rows = w * EMBED_DIM // LANE_PAD
    r0 = pl.multiple_of(v0 * EMBED_DIM // LANE_PAD, 8)
    pltpu.sync_copy(buf_b.at[pl.ds(0, out_rows), :],
                    out_hbm.at[pl.ds(r0, out_rows), :])


def _transpose_body(tabt_hbm, out_hbm, buf_a, buf_b, sem):
    wid = lax.axis_index("s") * 2 + lax.axis_index("c")

    @pl.loop(wid, NBLK, step=NUM_WORKERS)
    def _blk(blk):
        _transpose_block(tabt_hbm, out_hbm, buf_a, buf_b, blk * TW, TW)

    @pl.when(wid == NUM_WORKERS - 1)
    def _tail():
        _transpose_block(tabt_hbm, out_hbm, buf_a, buf_b, NBLK * TW, TAIL)


def _gather_body(idx_hbm, table_hbm, out_hbm, idx0, idx1, rows0, rows1,
                 sem0, sem1):
    wid = lax.axis_index("s") * 2 + lax.axis_index("c")
    base = wid * Q_PER_W

    def _wait(rows_v, sem):
        pltpu.make_async_copy(table_hbm.at[pl.ds(0, CQ)], rows_v, sem).wait()

    def _start(c, idx_v, rows_v, sem):
        pltpu.sync_copy(idx_hbm.at[pl.ds(base + c * CQ, CQ)], idx_v)
        pltpu.async_copy(table_hbm.at[idx_v], rows_v, sem)

    def _store(c, rows_v):
        pltpu.sync_copy(
            rows_v, out_hbm.at[pl.ds(base + c * CQ, CQ), pl.ds(0, EMBED_DIM)]
        )

    _start(0, idx0, rows0, sem0)

    @pl.loop(0, NCHUNK, step=2)
    def _chunk(c):
        _start(c + 1, idx1, rows1, sem1)
        _wait(rows0, sem0)
        _store(c, rows0)

        @pl.when(c + 2 < NCHUNK)
        def _():
            _start(c + 2, idx0, rows0, sem0)

        _wait(rows1, sem1)
        _store(c + 1, rows1)


@jax.jit
def _embed(token_ids, embeddings):
    mesh = plsc.VectorSubcoreMesh(core_axis_name="c", subcore_axis_name="s")

    transpose_kernel = pl.kernel(
        _transpose_body,
        out_type=jax.ShapeDtypeStruct((ROWS_TOTAL, LANE_PAD), jnp.float32),
        mesh=mesh,
        scratch_types=[
            pltpu.VMEM((EMBED_DIM, TW), jnp.float32),
            pltpu.VMEM((TOUT_ROWS, LANE_PAD), jnp.float32),
            pltpu.SemaphoreType.DMA,
        ],
        compiler_params=pltpu.CompilerParams(needs_layout_passes=False),
    )
    table250 = transpose_kernel(embeddings.T)
    tail = lax.slice(embeddings, (V_COVERED, 0), (VOCAB_SIZE, EMBED_DIM))
    n_tail_rows = (VOCAB_SIZE - V_COVERED) * EMBED_DIM // LANE_PAD
    table250 = lax.dynamic_update_slice(
        table250, tail.reshape(n_tail_rows, LANE_PAD),
        (V_COVERED * EMBED_DIM // LANE_PAD, 0),
    )
    table_rm = table250.reshape(VOCAB_SIZE, EMBED_DIM)

    idx56 = jnp.concatenate(
        [token_ids, token_ids[:, : HIST_PAD - HIST]], axis=1
    ).astype(jnp.int32)
    idx_flat = idx56.reshape(-1)

    gather_kernel = pl.kernel(
        _gather_body,
        out_type=jax.ShapeDtypeStruct((Q_TOTAL, LANE_PAD), jnp.float32),
        mesh=mesh,
        scratch_types=[
            pltpu.VMEM((CQ,), jnp.int32),
            pltpu.VMEM((CQ,), jnp.int32),
            pltpu.VMEM((CQ, EMBED_DIM), jnp.float32),
            pltpu.VMEM((CQ, EMBED_DIM), jnp.float32),
            pltpu.SemaphoreType.DMA,
            pltpu.SemaphoreType.DMA,
        ],
        compiler_params=pltpu.CompilerParams(use_tc_tiling_on_sc=False),
    )
    padded = gather_kernel(idx_flat, table_rm)
    padded3 = padded.reshape(BATCH, HIST_PAD, LANE_PAD)
    return lax.slice(padded3, (0, 0, 0), (BATCH, HIST, EMBED_DIM))


def kernel(token_ids, embeddings):
    return _embed(token_ids, embeddings)


# speedup vs baseline: 2.4701x; 1.5875x over previous
import jax
import jax.numpy as jnp
from jax import lax
from jax.experimental import pallas as pl
from jax.experimental.pallas import tpu as pltpu
from jax.experimental.pallas import tpu_sc as plsc

VOCAB_SIZE = 1_000_000
EMBED_DIM = 32
BATCH = 16384
HIST = 50
HIST_PAD = 56
LANE_PAD = 128
Q_TOTAL = BATCH * HIST_PAD
NUM_WORKERS = 32
Q_PER_W = Q_TOTAL // NUM_WORKERS
CQ = 1792
NCHUNK = Q_PER_W // CQ


def _body(idx_hbm, table_hbm, out_hbm, idx0, idx1, rows0, rows1, sem0, sem1):
    wid = lax.axis_index("s") * 2 + lax.axis_index("c")
    base = wid * Q_PER_W

    def _wait(rows_v, sem):
        pltpu.make_async_copy(
            table_hbm.at[pl.ds(0, CQ)], rows_v, sem
        ).wait()

    def _start(c, idx_v, rows_v, sem):
        pltpu.sync_copy(idx_hbm.at[pl.ds(base + c * CQ, CQ)], idx_v)
        pltpu.async_copy(table_hbm.at[idx_v], rows_v, sem)

    def _store(c, rows_v):
        pltpu.sync_copy(
            rows_v, out_hbm.at[pl.ds(base + c * CQ, CQ), pl.ds(0, EMBED_DIM)]
        )

    _start(0, idx0, rows0, sem0)

    @pl.loop(0, NCHUNK, step=2)
    def _chunk(c):
        _start(c + 1, idx1, rows1, sem1)
        _wait(rows0, sem0)
        _store(c, rows0)

        @pl.when(c + 2 < NCHUNK)
        def _():
            _start(c + 2, idx0, rows0, sem0)

        _wait(rows1, sem1)
        _store(c + 1, rows1)


@jax.jit
def _embed(token_ids, embeddings):
    idx56 = jnp.concatenate(
        [token_ids, token_ids[:, : HIST_PAD - HIST]], axis=1
    ).astype(jnp.int32)
    idx_flat = idx56.reshape(-1)

    mesh = plsc.VectorSubcoreMesh(core_axis_name="c", subcore_axis_name="s")
    grid_kernel = pl.kernel(
        _body,
        out_type=jax.ShapeDtypeStruct((Q_TOTAL, LANE_PAD), jnp.float32),
        mesh=mesh,
        scratch_types=[
            pltpu.VMEM((CQ,), jnp.int32),
            pltpu.VMEM((CQ,), jnp.int32),
            pltpu.VMEM((CQ, EMBED_DIM), jnp.float32),
            pltpu.VMEM((CQ, EMBED_DIM), jnp.float32),
            pltpu.SemaphoreType.DMA,
            pltpu.SemaphoreType.DMA,
        ],
        compiler_params=pltpu.CompilerParams(use_tc_tiling_on_sc=False),
    )
    padded = grid_kernel(idx_flat, embeddings)
    padded3 = padded.reshape(BATCH, HIST_PAD, LANE_PAD)
    return lax.slice(padded3, (0, 0, 0), (BATCH, HIST, EMBED_DIM))


def kernel(token_ids, embeddings):
    return _embed(token_ids, embeddings)


# speedup vs baseline: 2.4846x; 1.0059x over previous
import jax
import jax.numpy as jnp
from jax import lax
from jax.experimental import pallas as pl
from jax.experimental.pallas import tpu as pltpu
from jax.experimental.pallas import tpu_sc as plsc

VOCAB_SIZE = 1_000_000
EMBED_DIM = 32
BATCH = 16384
HIST = 50
HIST_PAD = 56
LANE_PAD = 128
Q_TOTAL = BATCH * HIST_PAD
NUM_WORKERS = 32
Q_PER_W = Q_TOTAL // NUM_WORKERS
CQ = 3584
NCHUNK = Q_PER_W // CQ


def _body(idx_hbm, table_hbm, out_hbm, idx_v, rows_v, sem):
    wid = lax.axis_index("s") * 2 + lax.axis_index("c")
    base = wid * Q_PER_W

    @pl.loop(0, NCHUNK)
    def _chunk(c):
        q0 = base + c * CQ
        pltpu.sync_copy(idx_hbm.at[pl.ds(q0, CQ)], idx_v)
        pltpu.async_copy(table_hbm.at[idx_v], rows_v, sem).wait()
        pltpu.sync_copy(rows_v, out_hbm.at[pl.ds(q0, CQ), pl.ds(0, EMBED_DIM)])


@jax.jit
def _embed(token_ids, embeddings):
    idx56 = jnp.concatenate(
        [token_ids, token_ids[:, : HIST_PAD - HIST]], axis=1
    ).astype(jnp.int32)
    idx_flat = idx56.reshape(-1)

    mesh = plsc.VectorSubcoreMesh(core_axis_name="c", subcore_axis_name="s")
    grid_kernel = pl.kernel(
        _body,
        out_type=jax.ShapeDtypeStruct((Q_TOTAL, LANE_PAD), jnp.float32),
        mesh=mesh,
        scratch_types=[
            pltpu.VMEM((CQ,), jnp.int32),
            pltpu.VMEM((CQ, EMBED_DIM), jnp.float32),
            pltpu.SemaphoreType.DMA,
        ],
        compiler_params=pltpu.CompilerParams(use_tc_tiling_on_sc=False),
    )
    padded = grid_kernel(idx_flat, embeddings)
    padded3 = padded.reshape(BATCH, HIST_PAD, LANE_PAD)
    return lax.slice(padded3, (0, 0, 0), (BATCH, HIST, EMBED_DIM))


def kernel(token_ids, embeddings):
    return _embed(token_ids, embeddings)
